# baseline (speedup 1.0000x reference)
import functools

import jax
import jax.numpy as jnp
from jax import lax
from jax.experimental import pallas as pl
from jax.experimental.pallas import tpu as pltpu
from jax.experimental.pallas import tpu_sc as plsc

B, T = 512, 17
SLOT_DIM, EMB_DIM, ACTION_DIM, NUM_ACTIONS = 64, 64, 32, 8192
BB = 128
NB = B // BB
R = BB * (T - 1)
KC = 2048
NK = NUM_ACTIONS // KC
NROWS = B * (T - 1)
NW = 32
CH = 128
CPW = NROWS // (NW * CH)


def _bmm(a, b):
    return lax.dot_general(a.astype(jnp.bfloat16), b.astype(jnp.bfloat16),
                           (((1,), (0,)), ((), ())),
                           preferred_element_type=jnp.float32)


def _front_body(slots_ref, noise_ref, we_ref, be_ref, wm_ref, bm_ref,
                wv_ref, bv_ref, z_ref):
    s_cat = jnp.concatenate([slots_ref[t] for t in range(T)], axis=0)
    tok = jnp.maximum(_bmm(s_cat, we_ref[...]) + be_ref[...], 0.0)
    mc = _bmm(tok, wm_ref[...]) + bm_ref[...]
    vc = jnp.abs(_bmm(tok, wv_ref[...]) + bv_ref[...])
    md = mc[BB:] - mc[:-BB]
    vs = vc[BB:] + vc[:-BB]
    n_cat = jnp.concatenate([noise_ref[t] for t in range(T - 1)], axis=0)
    z_ref[...] = n_cat * jnp.sqrt(vs + 1e-6) + md


def _dist_body(z_ref, zn_ref, cb_ref, cn_ref, idx_ref, bd_s, bi_s, io_s):
    k = pl.program_id(1)
    i = pl.program_id(0)

    @pl.when(jnp.logical_and(i == 0, k == 0))
    def _iota_init():
        io_s[...] = lax.broadcasted_iota(
            jnp.int32, (R, KC), 1).astype(jnp.float32)
    z2 = z_ref[...]
    m2 = lax.dot_general((z2 + z2).astype(jnp.bfloat16), cb_ref[...],
                         (((1,), (1,)), ((), ())),
                         preferred_element_type=jnp.float32)
    d = zn_ref[...] - m2 + cn_ref[...][None, :]
    dmin = jnp.min(d, axis=1, keepdims=True)
    cand = jnp.argmin(d, axis=1, keepdims=True).astype(jnp.int32) + k * KC
    dminq = dmin.astype(jnp.bfloat16).astype(jnp.float32)

    @pl.when(k == 0)
    def _first():
        bd_s[...] = dminq
        bi_s[...] = cand

    @pl.when(k != 0)
    def _rest():
        upd = dmin < bd_s[...]
        bi_s[...] = jnp.where(upd, cand, bi_s[...])
        bd_s[...] = jnp.where(upd, dminq, bd_s[...])

    @pl.when(k == NK - 1)
    def _finish():
        idx_ref[...] = bi_s[...]


def _sc_body(cb_hbm, idx_hbm, z_hbm, st_hbm, part_hbm,
             idx_v, q_v, zr_v, st_v, acc_v, sem1, sem2):
    wid = lax.axis_index("s") * 2 + lax.axis_index("c")
    acc = jnp.zeros((16,), jnp.float32)
    for cc in range(CPW):
        c = wid * CPW + cc
        pltpu.sync_copy(idx_hbm.at[c], idx_v)
        cp1 = pltpu.async_copy(cb_hbm.at[idx_v], q_v, sem1)
        cp2 = pltpu.async_copy(z_hbm.at[pl.ds(c * CH, CH)], zr_v, sem2)
        cp1.wait()
        cp2.wait()

        def row(r, a):
            z0 = zr_v[r, pl.ds(0, 16)]
            z1 = zr_v[r, pl.ds(16, 16)]
            q0 = q_v[r, pl.ds(0, 16)]
            q1 = q_v[r, pl.ds(16, 16)]
            st_v[r, pl.ds(0, 16)] = z0 + (q0 - z0)
            st_v[r, pl.ds(16, 16)] = z1 + (q1 - z1)
            d0 = z0 - q0
            d1 = z1 - q1
            return a + d0 * d0 + d1 * d1

        acc = lax.fori_loop(0, CH, row, acc)
        pltpu.sync_copy(st_v, st_hbm.at[pl.ds(c * CH, CH)])
    acc_v[...] = acc
    pltpu.sync_copy(acc_v, part_hbm.at[wid])


@jax.jit
def kernel(slots, noise, W_enc, b_enc, W_mean, b_mean, W_var, b_var, codebook):
    slots_t = jnp.transpose(slots, (1, 0, 2))
    noise_t = jnp.transpose(noise, (1, 0, 2))

    z_rows = pl.pallas_call(
        _front_body,
        grid=(NB,),
        in_specs=[
            pl.BlockSpec((T, BB, SLOT_DIM), lambda i: (0, i, 0)),
            pl.BlockSpec((T - 1, BB, ACTION_DIM), lambda i: (0, i, 0)),
            pl.BlockSpec((SLOT_DIM, EMB_DIM), lambda i: (0, 0)),
            pl.BlockSpec((EMB_DIM,), lambda i: (0,)),
            pl.BlockSpec((EMB_DIM, ACTION_DIM), lambda i: (0, 0)),
            pl.BlockSpec((ACTION_DIM,), lambda i: (0,)),
            pl.BlockSpec((EMB_DIM, ACTION_DIM), lambda i: (0, 0)),
            pl.BlockSpec((ACTION_DIM,), lambda i: (0,)),
        ],
        out_specs=pl.BlockSpec((R, ACTION_DIM), lambda i: (i, 0)),
        out_shape=jax.ShapeDtypeStruct((NROWS, ACTION_DIM), jnp.float32),
        compiler_params=pltpu.CompilerParams(
            dimension_semantics=("arbitrary",)),
    )(slots_t, noise_t, W_enc, b_enc, W_mean, b_mean, W_var, b_var)

    zn = jnp.sum(z_rows ** 2, axis=1, keepdims=True)
    cn = jnp.sum(codebook ** 2, axis=1)

    idx_rows = pl.pallas_call(
        _dist_body,
        grid=(NB, NK),
        in_specs=[
            pl.BlockSpec((R, ACTION_DIM), lambda i, k: (i, 0)),
            pl.BlockSpec((R, 1), lambda i, k: (i, 0)),
            pl.BlockSpec((KC, ACTION_DIM), lambda i, k: (k, 0)),
            pl.BlockSpec((KC,), lambda i, k: (k,)),
        ],
        out_specs=pl.BlockSpec((R, 1), lambda i, k: (i, 0)),
        out_shape=jax.ShapeDtypeStruct((NROWS, 1), jnp.int32),
        scratch_shapes=[
            pltpu.VMEM((R, 1), jnp.float32),
            pltpu.VMEM((R, 1), jnp.int32),
            pltpu.VMEM((R, KC), jnp.float32),
        ],
        compiler_params=pltpu.CompilerParams(
            dimension_semantics=("arbitrary", "arbitrary")),
    )(z_rows, zn, codebook, cn)

    idx = idx_rows.reshape(NB, T - 1, BB).transpose(0, 2, 1).reshape(B, T - 1)

    cb_pad = jnp.pad(codebook, ((0, 0), (0, 128 - ACTION_DIM)))

    sc = pl.kernel(
        _sc_body,
        mesh=plsc.VectorSubcoreMesh(core_axis_name="c", subcore_axis_name="s"),
        out_type=[
            jax.ShapeDtypeStruct((NROWS, ACTION_DIM), jnp.float32),
            jax.ShapeDtypeStruct((NW, 16), jnp.float32),
        ],
        scratch_types=[
            pltpu.VMEM((CH,), jnp.int32),
            pltpu.VMEM((CH, 128), jnp.float32),
            pltpu.VMEM((CH, ACTION_DIM), jnp.float32),
            pltpu.VMEM((CH, ACTION_DIM), jnp.float32),
            pltpu.VMEM((16,), jnp.float32),
            pltpu.SemaphoreType.DMA,
            pltpu.SemaphoreType.DMA,
        ],
    )
    st_tm, part = sc(cb_pad, idx_rows.reshape(NROWS // CH, CH), z_rows)

    st = st_tm.reshape(NB, T - 1, BB, ACTION_DIM).transpose(0, 2, 1, 3)
    st = st.reshape(B, T - 1, ACTION_DIM)
    l = jnp.sum(part) / (B * (T - 1) * ACTION_DIM)
    vq_loss = l + 0.25 * l
    return st, idx, vq_loss


# speedup vs baseline: 1.2656x; 1.2656x over previous
import functools

import jax
import jax.numpy as jnp
from jax import lax
from jax.experimental import pallas as pl
from jax.experimental.pallas import tpu as pltpu
from jax.experimental.pallas import tpu_sc as plsc

B, T = 512, 17
SLOT_DIM, EMB_DIM, ACTION_DIM, NUM_ACTIONS = 64, 64, 32, 8192
BB = 128
NB = B // BB
R = BB * (T - 1)
KC = 2048
NK = NUM_ACTIONS // KC
NROWS = B * (T - 1)
NW = 32
CH = 128
CPW = NROWS // (NW * CH)


def _bmm(a, b):
    return lax.dot_general(a.astype(jnp.bfloat16), b.astype(jnp.bfloat16),
                           (((1,), (0,)), ((), ())),
                           preferred_element_type=jnp.float32)


def _front_body(slots_ref, noise_ref, we_ref, be_ref, wm_ref, bm_ref,
                wv_ref, bv_ref, z_ref):
    s_cat = jnp.concatenate([slots_ref[t] for t in range(T)], axis=0)
    tok = jnp.maximum(_bmm(s_cat, we_ref[...]) + be_ref[...], 0.0)
    mc = _bmm(tok, wm_ref[...]) + bm_ref[...]
    vc = jnp.abs(_bmm(tok, wv_ref[...]) + bv_ref[...])
    md = mc[BB:] - mc[:-BB]
    vs = vc[BB:] + vc[:-BB]
    n_cat = jnp.concatenate([noise_ref[t] for t in range(T - 1)], axis=0)
    z_ref[...] = n_cat * jnp.sqrt(vs + 1e-6) + md


def _dist_body(z_ref, zn_ref, cb_ref, cn_ref, idx_ref, bd_s, bi_s, io_s):
    k = pl.program_id(1)
    i = pl.program_id(0)

    @pl.when(jnp.logical_and(i == 0, k == 0))
    def _iota_init():
        io_s[...] = lax.broadcasted_iota(
            jnp.int32, (R, KC), 1).astype(jnp.float32)
    z2 = z_ref[...]
    m2 = lax.dot_general((z2 + z2).astype(jnp.bfloat16), cb_ref[...],
                         (((1,), (1,)), ((), ())),
                         preferred_element_type=jnp.float32)
    d = zn_ref[...] - m2 + cn_ref[...][None, :]
    dmin = jnp.min(d, axis=1, keepdims=True)
    iota_f = lax.broadcasted_iota(jnp.int32, (R, KC), 1).astype(jnp.float32)
    cand = jnp.min(jnp.where(d == dmin, iota_f, jnp.float32(1e9)),
                   axis=1, keepdims=True).astype(jnp.int32) + k * KC
    dminq = dmin.astype(jnp.bfloat16).astype(jnp.float32)

    @pl.when(k == 0)
    def _first():
        bd_s[...] = dminq
        bi_s[...] = cand

    @pl.when(k != 0)
    def _rest():
        upd = dmin < bd_s[...]
        bi_s[...] = jnp.where(upd, cand, bi_s[...])
        bd_s[...] = jnp.where(upd, dminq, bd_s[...])

    @pl.when(k == NK - 1)
    def _finish():
        idx_ref[...] = bi_s[...]


def _sc_body(cb_hbm, idx_hbm, z_hbm, st_hbm, part_hbm,
             idx_v, q_v, zr_v, st_v, acc_v, sem1, sem2):
    wid = lax.axis_index("s") * 2 + lax.axis_index("c")
    acc = jnp.zeros((16,), jnp.float32)
    for cc in range(CPW):
        c = wid * CPW + cc
        pltpu.sync_copy(idx_hbm.at[c], idx_v)
        cp1 = pltpu.async_copy(cb_hbm.at[idx_v], q_v, sem1)
        cp2 = pltpu.async_copy(z_hbm.at[pl.ds(c * CH, CH)], zr_v, sem2)
        cp1.wait()
        cp2.wait()

        def row(r, a):
            z0 = zr_v[r, pl.ds(0, 16)]
            z1 = zr_v[r, pl.ds(16, 16)]
            q0 = q_v[r, pl.ds(0, 16)]
            q1 = q_v[r, pl.ds(16, 16)]
            st_v[r, pl.ds(0, 16)] = z0 + (q0 - z0)
            st_v[r, pl.ds(16, 16)] = z1 + (q1 - z1)
            d0 = z0 - q0
            d1 = z1 - q1
            return a + d0 * d0 + d1 * d1

        acc = lax.fori_loop(0, CH, row, acc)
        pltpu.sync_copy(st_v, st_hbm.at[pl.ds(c * CH, CH)])
    acc_v[...] = acc
    pltpu.sync_copy(acc_v, part_hbm.at[wid])


@jax.jit
def kernel(slots, noise, W_enc, b_enc, W_mean, b_mean, W_var, b_var, codebook):
    slots_t = jnp.transpose(slots, (1, 0, 2))
    noise_t = jnp.transpose(noise, (1, 0, 2))

    z_rows = pl.pallas_call(
        _front_body,
        grid=(NB,),
        in_specs=[
            pl.BlockSpec((T, BB, SLOT_DIM), lambda i: (0, i, 0)),
            pl.BlockSpec((T - 1, BB, ACTION_DIM), lambda i: (0, i, 0)),
            pl.BlockSpec((SLOT_DIM, EMB_DIM), lambda i: (0, 0)),
            pl.BlockSpec((EMB_DIM,), lambda i: (0,)),
            pl.BlockSpec((EMB_DIM, ACTION_DIM), lambda i: (0, 0)),
            pl.BlockSpec((ACTION_DIM,), lambda i: (0,)),
            pl.BlockSpec((EMB_DIM, ACTION_DIM), lambda i: (0, 0)),
            pl.BlockSpec((ACTION_DIM,), lambda i: (0,)),
        ],
        out_specs=pl.BlockSpec((R, ACTION_DIM), lambda i: (i, 0)),
        out_shape=jax.ShapeDtypeStruct((NROWS, ACTION_DIM), jnp.float32),
        compiler_params=pltpu.CompilerParams(
            dimension_semantics=("arbitrary",)),
    )(slots_t, noise_t, W_enc, b_enc, W_mean, b_mean, W_var, b_var)

    zn = jnp.sum(z_rows ** 2, axis=1, keepdims=True)
    cn = jnp.sum(codebook ** 2, axis=1)

    idx_rows = pl.pallas_call(
        _dist_body,
        grid=(NB, NK),
        in_specs=[
            pl.BlockSpec((R, ACTION_DIM), lambda i, k: (i, 0)),
            pl.BlockSpec((R, 1), lambda i, k: (i, 0)),
            pl.BlockSpec((KC, ACTION_DIM), lambda i, k: (k, 0)),
            pl.BlockSpec((KC,), lambda i, k: (k,)),
        ],
        out_specs=pl.BlockSpec((R, 1), lambda i, k: (i, 0)),
        out_shape=jax.ShapeDtypeStruct((NROWS, 1), jnp.int32),
        scratch_shapes=[
            pltpu.VMEM((R, 1), jnp.float32),
            pltpu.VMEM((R, 1), jnp.int32),
            pltpu.VMEM((R, KC), jnp.float32),
        ],
        compiler_params=pltpu.CompilerParams(
            dimension_semantics=("arbitrary", "arbitrary")),
    )(z_rows, zn, codebook, cn)

    idx = idx_rows.reshape(NB, T - 1, BB).transpose(0, 2, 1).reshape(B, T - 1)

    cb_pad = jnp.pad(codebook, ((0, 0), (0, 128 - ACTION_DIM)))

    sc = pl.kernel(
        _sc_body,
        mesh=plsc.VectorSubcoreMesh(core_axis_name="c", subcore_axis_name="s"),
        out_type=[
            jax.ShapeDtypeStruct((NROWS, ACTION_DIM), jnp.float32),
            jax.ShapeDtypeStruct((NW, 16), jnp.float32),
        ],
        scratch_types=[
            pltpu.VMEM((CH,), jnp.int32),
            pltpu.VMEM((CH, 128), jnp.float32),
            pltpu.VMEM((CH, ACTION_DIM), jnp.float32),
            pltpu.VMEM((CH, ACTION_DIM), jnp.float32),
            pltpu.VMEM((16,), jnp.float32),
            pltpu.SemaphoreType.DMA,
            pltpu.SemaphoreType.DMA,
        ],
    )
    st_tm, part = sc(cb_pad, idx_rows.reshape(NROWS // CH, CH), z_rows)

    st = st_tm.reshape(NB, T - 1, BB, ACTION_DIM).transpose(0, 2, 1, 3)
    st = st.reshape(B, T - 1, ACTION_DIM)
    l = jnp.sum(part) / (B * (T - 1) * ACTION_DIM)
    vq_loss = l + 0.25 * l
    return st, idx, vq_loss
